# baseline (speedup 1.0000x reference)
import functools

import jax
import jax.numpy as jnp
from jax import lax
from jax.experimental import pallas as pl
from jax.experimental.pallas import tpu as pltpu
from jax.experimental.pallas import tpu_sc as plsc

_B = 64
_NBUF = 4
_NIDX = 8


def _make_sc_aggregate(NP, D, EP_W, NB, NC, NS):
  rows_per_tile = NP // NS
  n_zero_blocks = rows_per_tile // 128
  mesh = plsc.VectorSubcoreMesh(core_axis_name="c", subcore_axis_name="s")

  @functools.partial(
      pl.kernel,
      out_type=(
          jax.ShapeDtypeStruct((NP, D), jnp.float32),
          jax.ShapeDtypeStruct((NP, D), jnp.float32),
          jax.ShapeDtypeStruct((NP,), jnp.float32),
          jax.ShapeDtypeStruct((NP,), jnp.float32),
      ),
      mesh=mesh,
      scratch_types=(
          [pltpu.VMEM((_B,), jnp.int32)] * _NIDX
          + [pltpu.VMEM((_B,), jnp.int32)] * _NIDX
          + [pltpu.VMEM((_B, D), jnp.float32)] * _NBUF
          + [
              pltpu.VMEM((64, D), jnp.float32),
              pltpu.VMEM((_B,), jnp.float32),
              pltpu.VMEM_SHARED((NP, D), jnp.float32),
              pltpu.VMEM_SHARED((NP,), jnp.float32),
          ]
          + [pltpu.SemaphoreType.DMA] * _NBUF
          + [pltpu.SemaphoreType.DMA] * _NIDX
          + [pltpu.SemaphoreType.DMA] * _NIDX
      ),
  )
  def sc_kernel(src_hbm, dst_hbm, x2_hbm, z2_hbm, ones_hbm,
                acc0_hbm, acc1_hbm, deg0_hbm, deg1_hbm, *refs):
    srcs = refs[:_NIDX]
    dsts = refs[_NIDX:2 * _NIDX]
    bufs = refs[2 * _NIDX:2 * _NIDX + _NBUF]
    zb, ones_v, acc_s, deg_s = refs[2 * _NIDX + _NBUF:2 * _NIDX + _NBUF + 4]
    sems = refs[2 * _NIDX + _NBUF + 4:]
    semg = sems[:_NBUF]
    semsrc = sems[_NBUF:_NBUF + _NIDX]
    semdst = sems[_NBUF + _NIDX:]

    cid = lax.axis_index("c")
    sid = lax.axis_index("s")
    wid = sid * NC + cid
    row0 = sid * rows_per_tile
    base = wid * EP_W

    def idx_wait(vref, sem):
      pltpu.make_async_copy(src_hbm.at[pl.ds(0, _B)], vref, sem).wait()

    def gather_wait(r):
      pltpu.make_async_copy(x2_hbm.at[0].at[pl.ds(0, _B)], bufs[r],
                            semg[r]).wait()

    pltpu.sync_copy(ones_hbm, ones_v)
    pltpu.sync_copy(z2_hbm, zb)
    for r in range(rows_per_tile // 64):
      pltpu.sync_copy(zb, acc_s.at[pl.ds(row0 + r * 64, 64)])
    for r in range(n_zero_blocks):
      pltpu.sync_copy(zb.at[0], deg_s.at[pl.ds(row0 + r * 128, 128)])
    plsc.subcore_barrier()

    for k in range(_NIDX):
      pltpu.async_copy(src_hbm.at[pl.ds(base + k * _B, _B)], srcs[k],
                       semsrc[k])
      pltpu.async_copy(dst_hbm.at[pl.ds(base + k * _B, _B)], dsts[k],
                       semdst[k])
    def gather_issue(idx_ref, buf_ref, sem):
      @pl.when(cid == 0)
      def _():
        pltpu.async_copy(x2_hbm.at[0].at[idx_ref], buf_ref, sem)

      @pl.when(cid == 1)
      def _():
        pltpu.async_copy(x2_hbm.at[1].at[idx_ref], buf_ref, sem)

    for k in range(_NBUF):
      idx_wait(srcs[k], semsrc[k])
      gather_issue(srcs[k], bufs[k], semg[k])

    @pl.loop(0, NB, step=_NIDX)
    def _(j):
      for b in range(_NIDX):
        jj = j + b
        r = b % _NBUF
        gather_wait(r)
        idx_wait(dsts[b], semdst[b])
        pltpu.sync_copy(bufs[r], acc_s.at[dsts[b]], add=True)
        pltpu.sync_copy(ones_v, deg_s.at[dsts[b]], add=True)

        @pl.when(jj + _NIDX < NB)
        def _():
          pltpu.async_copy(
              src_hbm.at[pl.ds(base + (jj + _NIDX) * _B, _B)], srcs[b],
              semsrc[b])
          pltpu.async_copy(
              dst_hbm.at[pl.ds(base + (jj + _NIDX) * _B, _B)], dsts[b],
              semdst[b])

        @pl.when(jj + _NBUF < NB)
        def _():
          o = (b + _NBUF) % _NIDX
          idx_wait(srcs[o], semsrc[o])
          gather_issue(srcs[o], bufs[r], semg[r])

    plsc.subcore_barrier()

    @pl.when(cid == 0)
    def _():
      pltpu.sync_copy(acc_s.at[pl.ds(row0, rows_per_tile)],
                      acc0_hbm.at[pl.ds(row0, rows_per_tile)])
      pltpu.sync_copy(deg_s.at[pl.ds(row0, rows_per_tile)],
                      deg0_hbm.at[pl.ds(row0, rows_per_tile)])

    @pl.when(cid == 1)
    def _():
      pltpu.sync_copy(acc_s.at[pl.ds(row0, rows_per_tile)],
                      acc1_hbm.at[pl.ds(row0, rows_per_tile)])
      pltpu.sync_copy(deg_s.at[pl.ds(row0, rows_per_tile)],
                      deg1_hbm.at[pl.ds(row0, rows_per_tile)])

  return sc_kernel


def _tc_finish(acc0_ref, acc1_ref, deg0_ref, deg1_ref, x_ref, wl_ref, wr_ref,
               b_ref, out_ref):
  deg = jnp.maximum(deg0_ref[...] + deg1_ref[...], 1.0)
  agg = (acc0_ref[...] + acc1_ref[...]) / deg
  out = (jnp.dot(agg, wl_ref[...], preferred_element_type=jnp.float32)
         + jnp.dot(x_ref[...], wr_ref[...], preferred_element_type=jnp.float32)
         + b_ref[...])
  norm = jnp.sqrt(jnp.sum(out * out, axis=1, keepdims=True))
  out = out / jnp.maximum(norm, 1e-12)
  out_ref[...] = jnp.maximum(out, 0.0)


def kernel(x, edge_index, batch, W_l, W_r, b):
  del batch
  N, D = x.shape
  E = edge_index.shape[1]
  NC, NS = 2, 16
  NW = NC * NS

  NP = ((N + 1 + NS * 128 - 1) // (NS * 128)) * (NS * 128)
  grp = NW * _B * _NIDX
  E_pad = ((E + grp - 1) // grp) * grp
  EP_W = E_pad // NW
  NB = EP_W // _B

  src = jnp.concatenate(
      [edge_index[0], jnp.zeros((E_pad - E,), jnp.int32)])
  dst = jnp.concatenate(
      [edge_index[1], jnp.full((E_pad - E,), N, jnp.int32)])
  x_pad = jnp.pad(x, ((0, NP - N), (0, 0)))
  z2 = jnp.zeros((64, D), jnp.float32)
  ones = jnp.ones((_B,), jnp.float32)

  x2 = jnp.stack([x_pad, x_pad])

  sc = _make_sc_aggregate(NP, D, EP_W, NB, NC, NS)
  acc0, acc1, deg0, deg1 = sc(src, dst, x2, z2, ones)

  R = 512
  grid = (NP // R,)
  out = pl.pallas_call(
      _tc_finish,
      grid=grid,
      in_specs=[
          pl.BlockSpec((R, D), lambda i: (i, 0)),
          pl.BlockSpec((R, D), lambda i: (i, 0)),
          pl.BlockSpec((R, 1), lambda i: (i, 0)),
          pl.BlockSpec((R, 1), lambda i: (i, 0)),
          pl.BlockSpec((R, D), lambda i: (i, 0)),
          pl.BlockSpec((D, D), lambda i: (0, 0)),
          pl.BlockSpec((D, D), lambda i: (0, 0)),
          pl.BlockSpec((1, D), lambda i: (0, 0)),
      ],
      out_specs=pl.BlockSpec((R, D), lambda i: (i, 0)),
      out_shape=jax.ShapeDtypeStruct((NP, D), jnp.float32),
  )(acc0, acc1, deg0.reshape(NP, 1), deg1.reshape(NP, 1), x_pad, W_l, W_r,
    b.reshape(1, D))
  return out[:N]


# speedup vs baseline: 1.3965x; 1.3965x over previous
import functools

import jax
import jax.numpy as jnp
from jax import lax
from jax.experimental import pallas as pl
from jax.experimental.pallas import tpu as pltpu
from jax.experimental.pallas import tpu_sc as plsc

_B = 64
_NBUF = 4
_NIDX = 8


def _make_sc_aggregate(NP, D, EPP, EP0, NB0, NB1, NC, NS):
  rows_per_tile = NP // NS
  n_zero_blocks = rows_per_tile // 128
  mesh = plsc.VectorSubcoreMesh(core_axis_name="c", subcore_axis_name="s")

  @functools.partial(
      pl.kernel,
      out_type=(
          jax.ShapeDtypeStruct((NP, D), jnp.float32),
          jax.ShapeDtypeStruct((NP, D), jnp.float32),
          jax.ShapeDtypeStruct((NP,), jnp.float32),
          jax.ShapeDtypeStruct((NP,), jnp.float32),
      ),
      mesh=mesh,
      scratch_types=(
          [pltpu.VMEM((_B,), jnp.int32)] * _NIDX
          + [pltpu.VMEM((_B,), jnp.int32)] * _NIDX
          + [pltpu.VMEM((_B, D), jnp.float32)] * _NBUF
          + [
              pltpu.VMEM((64, D), jnp.float32),
              pltpu.VMEM((_B,), jnp.float32),
              pltpu.VMEM_SHARED((NP, D), jnp.float32),
              pltpu.VMEM_SHARED((NP,), jnp.float32),
          ]
          + [pltpu.SemaphoreType.DMA] * _NBUF
          + [pltpu.SemaphoreType.DMA] * _NIDX
          + [pltpu.SemaphoreType.DMA] * _NIDX
      ),
  )
  def sc_kernel(src_hbm, dst_hbm, x_hbm, z2_hbm, ones_hbm,
                acc0_hbm, acc1_hbm, deg0_hbm, deg1_hbm, *refs):
    srcs = refs[:_NIDX]
    dsts = refs[_NIDX:2 * _NIDX]
    bufs = refs[2 * _NIDX:2 * _NIDX + _NBUF]
    zb, ones_v, acc_s, deg_s = refs[2 * _NIDX + _NBUF:2 * _NIDX + _NBUF + 4]
    sems = refs[2 * _NIDX + _NBUF + 4:]
    semg = sems[:_NBUF]
    semsrc = sems[_NBUF:_NBUF + _NIDX]
    semdst = sems[_NBUF + _NIDX:]

    cid = lax.axis_index("c")
    sid = lax.axis_index("s")
    row0 = sid * rows_per_tile
    base = sid * EPP + cid * EP0
    NBc = NB0 + cid * (NB1 - NB0)

    def idx_wait(vref, sem):
      pltpu.make_async_copy(src_hbm.at[pl.ds(0, _B)], vref, sem).wait()

    def gather_wait(r):
      pltpu.make_async_copy(x_hbm.at[pl.ds(0, _B)], bufs[r], semg[r]).wait()

    pltpu.sync_copy(ones_hbm, ones_v)
    pltpu.sync_copy(z2_hbm, zb)
    for r in range(rows_per_tile // 64):
      pltpu.sync_copy(zb, acc_s.at[pl.ds(row0 + r * 64, 64)])
    for r in range(n_zero_blocks):
      pltpu.sync_copy(zb.at[0], deg_s.at[pl.ds(row0 + r * 128, 128)])
    plsc.subcore_barrier()

    for k in range(_NIDX):
      pltpu.async_copy(src_hbm.at[pl.ds(base + k * _B, _B)], srcs[k],
                       semsrc[k])
      pltpu.async_copy(dst_hbm.at[pl.ds(base + k * _B, _B)], dsts[k],
                       semdst[k])
    for k in range(_NBUF):
      idx_wait(srcs[k], semsrc[k])
      pltpu.async_copy(x_hbm.at[srcs[k]], bufs[k], semg[k])

    @pl.loop(0, NBc, step=_NIDX)
    def _(j):
      for b in range(_NIDX):
        jj = j + b
        r = b % _NBUF
        gather_wait(r)
        idx_wait(dsts[b], semdst[b])
        pltpu.sync_copy(bufs[r], acc_s.at[dsts[b]], add=True)
        pltpu.sync_copy(ones_v, deg_s.at[dsts[b]], add=True)

        @pl.when(jj + _NIDX < NBc)
        def _():
          pltpu.async_copy(
              src_hbm.at[pl.ds(base + (jj + _NIDX) * _B, _B)], srcs[b],
              semsrc[b])
          pltpu.async_copy(
              dst_hbm.at[pl.ds(base + (jj + _NIDX) * _B, _B)], dsts[b],
              semdst[b])

        @pl.when(jj + _NBUF < NBc)
        def _():
          o = (b + _NBUF) % _NIDX
          idx_wait(srcs[o], semsrc[o])
          pltpu.async_copy(x_hbm.at[srcs[o]], bufs[r], semg[r])

    plsc.subcore_barrier()

    @pl.when(cid == 0)
    def _():
      pltpu.sync_copy(acc_s.at[pl.ds(row0, rows_per_tile)],
                      acc0_hbm.at[pl.ds(row0, rows_per_tile)])
      pltpu.sync_copy(deg_s.at[pl.ds(row0, rows_per_tile)],
                      deg0_hbm.at[pl.ds(row0, rows_per_tile)])

    @pl.when(cid == 1)
    def _():
      pltpu.sync_copy(acc_s.at[pl.ds(row0, rows_per_tile)],
                      acc1_hbm.at[pl.ds(row0, rows_per_tile)])
      pltpu.sync_copy(deg_s.at[pl.ds(row0, rows_per_tile)],
                      deg1_hbm.at[pl.ds(row0, rows_per_tile)])

  return sc_kernel


def _tc_finish(acc0_ref, acc1_ref, deg0_ref, deg1_ref, x_ref, wl_ref, wr_ref,
               b_ref, out_ref):
  deg = jnp.maximum(deg0_ref[...] + deg1_ref[...], 1.0)
  agg = (acc0_ref[...] + acc1_ref[...]) / deg
  out = (jnp.dot(agg, wl_ref[...], preferred_element_type=jnp.float32)
         + jnp.dot(x_ref[...], wr_ref[...], preferred_element_type=jnp.float32)
         + b_ref[...])
  norm = jnp.sqrt(jnp.sum(out * out, axis=1, keepdims=True))
  out = out / jnp.maximum(norm, 1e-12)
  out_ref[...] = jnp.maximum(out, 0.0)


def kernel(x, edge_index, batch, W_l, W_r, b):
  del batch
  N, D = x.shape
  E = edge_index.shape[1]
  NC, NS = 2, 16
  NW = NC * NS

  NP = ((N + 1 + NS * 128 - 1) // (NS * 128)) * (NS * 128)
  grp = NW * _B * _NIDX
  E_pad = ((E + grp - 1) // grp) * grp
  EPP = E_pad // NS
  gpp = EPP // (_B * _NIDX)
  g0 = max(1, min(gpp - 1, round(gpp * 0.8)))
  EP0 = g0 * _B * _NIDX
  NB0 = EP0 // _B
  NB1 = (EPP - EP0) // _B

  src = jnp.concatenate(
      [edge_index[0], jnp.zeros((E_pad - E,), jnp.int32)])
  dst = jnp.concatenate(
      [edge_index[1], jnp.full((E_pad - E,), N, jnp.int32)])
  x_pad = jnp.pad(x, ((0, NP - N), (0, 0)))
  z2 = jnp.zeros((64, D), jnp.float32)
  ones = jnp.ones((_B,), jnp.float32)

  sc = _make_sc_aggregate(NP, D, EPP, EP0, NB0, NB1, NC, NS)
  acc0, acc1, deg0, deg1 = sc(src, dst, x_pad, z2, ones)

  R = 512
  grid = (NP // R,)
  out = pl.pallas_call(
      _tc_finish,
      grid=grid,
      in_specs=[
          pl.BlockSpec((R, D), lambda i: (i, 0)),
          pl.BlockSpec((R, D), lambda i: (i, 0)),
          pl.BlockSpec((R, 1), lambda i: (i, 0)),
          pl.BlockSpec((R, 1), lambda i: (i, 0)),
          pl.BlockSpec((R, D), lambda i: (i, 0)),
          pl.BlockSpec((D, D), lambda i: (0, 0)),
          pl.BlockSpec((D, D), lambda i: (0, 0)),
          pl.BlockSpec((1, D), lambda i: (0, 0)),
      ],
      out_specs=pl.BlockSpec((R, D), lambda i: (i, 0)),
      out_shape=jax.ShapeDtypeStruct((NP, D), jnp.float32),
  )(acc0, acc1, deg0.reshape(NP, 1), deg1.reshape(NP, 1), x_pad, W_l, W_r,
    b.reshape(1, D))
  return out[:N]


# speedup vs baseline: 1.3966x; 1.0000x over previous
import functools

import jax
import jax.numpy as jnp
from jax import lax
from jax.experimental import pallas as pl
from jax.experimental.pallas import tpu as pltpu
from jax.experimental.pallas import tpu_sc as plsc

_B = 64
_NBUF = 4
_NIDX = 8


def _make_sc_aggregate(NP, D, EPP, EP0, NB0, NB1, NC, NS):
  rows_per_tile = NP // NS
  n_zero_blocks = rows_per_tile // 128
  mesh = plsc.VectorSubcoreMesh(core_axis_name="c", subcore_axis_name="s")

  @functools.partial(
      pl.kernel,
      out_type=(
          jax.ShapeDtypeStruct((NP, D), jnp.float32),
          jax.ShapeDtypeStruct((NP, D), jnp.float32),
          jax.ShapeDtypeStruct((NP,), jnp.float32),
          jax.ShapeDtypeStruct((NP,), jnp.float32),
      ),
      mesh=mesh,
      scratch_types=(
          [pltpu.VMEM((_B,), jnp.int32)] * _NIDX
          + [pltpu.VMEM((_B,), jnp.int32)] * _NIDX
          + [pltpu.VMEM((_B, D), jnp.float32)] * _NBUF
          + [
              pltpu.VMEM((64, D), jnp.float32),
              pltpu.VMEM((_B,), jnp.float32),
              pltpu.VMEM_SHARED((NP, D), jnp.float32),
              pltpu.VMEM_SHARED((NP,), jnp.float32),
          ]
          + [pltpu.SemaphoreType.DMA] * _NBUF
          + [pltpu.SemaphoreType.DMA] * _NIDX
          + [pltpu.SemaphoreType.DMA] * _NIDX
      ),
  )
  def sc_kernel(src_hbm, dst_hbm, x_hbm, z2_hbm, ones_hbm,
                acc0_hbm, acc1_hbm, deg0_hbm, deg1_hbm, *refs):
    srcs = refs[:_NIDX]
    dsts = refs[_NIDX:2 * _NIDX]
    bufs = refs[2 * _NIDX:2 * _NIDX + _NBUF]
    zb, ones_v, acc_s, deg_s = refs[2 * _NIDX + _NBUF:2 * _NIDX + _NBUF + 4]
    sems = refs[2 * _NIDX + _NBUF + 4:]
    semg = sems[:_NBUF]
    semsrc = sems[_NBUF:_NBUF + _NIDX]
    semdst = sems[_NBUF + _NIDX:]

    cid = lax.axis_index("c")
    sid = lax.axis_index("s")
    row0 = sid * rows_per_tile
    base = sid * EPP + cid * EP0

    def idx_wait(vref, sem):
      pltpu.make_async_copy(src_hbm.at[pl.ds(0, _B)], vref, sem).wait()

    def gather_wait(r):
      pltpu.make_async_copy(x_hbm.at[pl.ds(0, _B)], bufs[r], semg[r]).wait()

    pltpu.sync_copy(ones_hbm, ones_v)
    pltpu.sync_copy(z2_hbm, zb)
    for r in range(rows_per_tile // 64):
      pltpu.sync_copy(zb, acc_s.at[pl.ds(row0 + r * 64, 64)])
    for r in range(n_zero_blocks):
      pltpu.sync_copy(zb.at[0], deg_s.at[pl.ds(row0 + r * 128, 128)])
    plsc.subcore_barrier()

    for k in range(_NIDX):
      pltpu.async_copy(src_hbm.at[pl.ds(base + k * _B, _B)], srcs[k],
                       semsrc[k])
      pltpu.async_copy(dst_hbm.at[pl.ds(base + k * _B, _B)], dsts[k],
                       semdst[k])
    for k in range(_NBUF):
      idx_wait(srcs[k], semsrc[k])
      pltpu.async_copy(x_hbm.at[srcs[k]], bufs[k], semg[k])

    def edge_loop(nb):
      @pl.loop(0, nb, step=_NIDX)
      def _(j):
        for b in range(_NIDX):
          jj = j + b
          r = b % _NBUF
          gather_wait(r)
          idx_wait(dsts[b], semdst[b])
          pltpu.sync_copy(bufs[r], acc_s.at[dsts[b]], add=True)
          pltpu.sync_copy(ones_v, deg_s.at[dsts[b]], add=True)

          @pl.when(jj + _NIDX < nb)
          def _():
            pltpu.async_copy(
                src_hbm.at[pl.ds(base + (jj + _NIDX) * _B, _B)], srcs[b],
                semsrc[b])
            pltpu.async_copy(
                dst_hbm.at[pl.ds(base + (jj + _NIDX) * _B, _B)], dsts[b],
                semdst[b])

          @pl.when(jj + _NBUF < nb)
          def _():
            o = (b + _NBUF) % _NIDX
            idx_wait(srcs[o], semsrc[o])
            pltpu.async_copy(x_hbm.at[srcs[o]], bufs[r], semg[r])

    @pl.when(cid == 0)
    def _():
      edge_loop(NB0)

    @pl.when(cid == 1)
    def _():
      edge_loop(NB1)

    plsc.subcore_barrier()

    @pl.when(cid == 0)
    def _():
      pltpu.sync_copy(acc_s.at[pl.ds(row0, rows_per_tile)],
                      acc0_hbm.at[pl.ds(row0, rows_per_tile)])
      pltpu.sync_copy(deg_s.at[pl.ds(row0, rows_per_tile)],
                      deg0_hbm.at[pl.ds(row0, rows_per_tile)])

    @pl.when(cid == 1)
    def _():
      pltpu.sync_copy(acc_s.at[pl.ds(row0, rows_per_tile)],
                      acc1_hbm.at[pl.ds(row0, rows_per_tile)])
      pltpu.sync_copy(deg_s.at[pl.ds(row0, rows_per_tile)],
                      deg1_hbm.at[pl.ds(row0, rows_per_tile)])

  return sc_kernel


def _tc_finish(acc0_ref, acc1_ref, deg0_ref, deg1_ref, x_ref, wl_ref, wr_ref,
               b_ref, out_ref):
  deg = jnp.maximum(deg0_ref[...] + deg1_ref[...], 1.0)
  agg = (acc0_ref[...] + acc1_ref[...]) / deg
  out = (jnp.dot(agg, wl_ref[...], preferred_element_type=jnp.float32)
         + jnp.dot(x_ref[...], wr_ref[...], preferred_element_type=jnp.float32)
         + b_ref[...])
  norm = jnp.sqrt(jnp.sum(out * out, axis=1, keepdims=True))
  out = out / jnp.maximum(norm, 1e-12)
  out_ref[...] = jnp.maximum(out, 0.0)


def kernel(x, edge_index, batch, W_l, W_r, b):
  del batch
  N, D = x.shape
  E = edge_index.shape[1]
  NC, NS = 2, 16
  NW = NC * NS

  NP = ((N + 1 + NS * 128 - 1) // (NS * 128)) * (NS * 128)
  grp = NW * _B * _NIDX
  E_pad = ((E + grp - 1) // grp) * grp
  EPP = E_pad // NS
  gpp = EPP // (_B * _NIDX)
  g0 = max(1, min(gpp - 1, round(gpp * 0.8)))
  EP0 = g0 * _B * _NIDX
  NB0 = EP0 // _B
  NB1 = (EPP - EP0) // _B

  src = jnp.concatenate(
      [edge_index[0], jnp.zeros((E_pad - E,), jnp.int32)])
  dst = jnp.concatenate(
      [edge_index[1], jnp.full((E_pad - E,), N, jnp.int32)])
  x_pad = jnp.pad(x, ((0, NP - N), (0, 0)))
  z2 = jnp.zeros((64, D), jnp.float32)
  ones = jnp.ones((_B,), jnp.float32)

  sc = _make_sc_aggregate(NP, D, EPP, EP0, NB0, NB1, NC, NS)
  acc0, acc1, deg0, deg1 = sc(src, dst, x_pad, z2, ones)

  R = 512
  grid = (NP // R,)
  out = pl.pallas_call(
      _tc_finish,
      grid=grid,
      in_specs=[
          pl.BlockSpec((R, D), lambda i: (i, 0)),
          pl.BlockSpec((R, D), lambda i: (i, 0)),
          pl.BlockSpec((R, 1), lambda i: (i, 0)),
          pl.BlockSpec((R, 1), lambda i: (i, 0)),
          pl.BlockSpec((R, D), lambda i: (i, 0)),
          pl.BlockSpec((D, D), lambda i: (0, 0)),
          pl.BlockSpec((D, D), lambda i: (0, 0)),
          pl.BlockSpec((1, D), lambda i: (0, 0)),
      ],
      out_specs=pl.BlockSpec((R, D), lambda i: (i, 0)),
      out_shape=jax.ShapeDtypeStruct((NP, D), jnp.float32),
  )(acc0, acc1, deg0.reshape(NP, 1), deg1.reshape(NP, 1), x_pad, W_l, W_r,
    b.reshape(1, D))
  return out[:N]
